# baseline (speedup 1.0000x reference)
import numpy as np
import jax
import jax.numpy as jnp
from jax.experimental import pallas as pl

_V_PATCH = (1, 2, 3, 4, 5, 6, 8, 10, 13, 16)
_RATIO = 0.5
_BETA = 0.25
_K = 8192
_Bb, _Tt, _Hh, _Ww, _Cc = 8, 4, 16, 16, 32
_SN = len(_V_PATCH)
_BLK = 256
_DEF = jax.lax.Precision.DEFAULT


def _phi_k(si):
    k = 4
    ticks = np.linspace(1.0 / 3 / k, 1 - 1.0 / 3 / k, k)
    return int(np.argmin(np.abs(ticks - si / (_SN - 1))))


_KK = tuple(_phi_k(si) for si in range(_SN))


def _dist_body(rest_ref, rest2_ref, embT_ref, embsq_ref, idx_ref):
    blk = rest_ref[...]
    scores = jnp.dot(blk, embT_ref[...], precision=_DEF)
    d = (rest2_ref[...] + embsq_ref[...]) - 2.0 * scores
    ii = jnp.argmin(d, axis=1).astype(jnp.int32)
    idx_ref[...] = ii.reshape(_BLK, 1)


def _nearest(rest_nc, embT, embsq):
    n = rest_nc.shape[0]
    npad = 2048 if n <= 2048 else 8192
    rest2 = jnp.sum(rest_nc * rest_nc, axis=1, keepdims=True)
    restP = jnp.pad(rest_nc, ((0, npad - n), (0, 0)))
    rest2P = jnp.pad(rest2, ((0, npad - n), (0, 0)))
    grid = (npad // _BLK,)
    idx = pl.pallas_call(
        _dist_body,
        grid=grid,
        in_specs=[pl.BlockSpec((_BLK, _Cc), lambda i: (i, 0)),
                  pl.BlockSpec((_BLK, 1), lambda i: (i, 0)),
                  pl.BlockSpec((_Cc, _K), lambda i: (0, 0)),
                  pl.BlockSpec((1, _K), lambda i: (0, 0))],
        out_specs=pl.BlockSpec((_BLK, 1), lambda i: (i, 0)),
        out_shape=jax.ShapeDtypeStruct((npad, 1), jnp.int32),
    )(restP, rest2P, embT, embsq)
    return idx[:n, 0]


def kernel(f_BCThw, emb, phi_w, phi_b):
    f = f_BCThw.astype(jnp.float32)
    embT = emb.T
    embsq = jnp.sum(emb * emb, axis=1).reshape(1, _K)

    f_rest = f
    f_hat = jnp.zeros_like(f)
    loss = jnp.float32(0.0)
    idx_list = []
    ratio = abs(_RATIO)
    for si, pn in enumerate(_V_PATCH):
        if si != _SN - 1:
            rest = jax.image.resize(f_rest, (_Bb, _Cc, _Tt, pn, pn),
                                    method='trilinear')
        else:
            rest = f_rest
        rest_nc = jnp.transpose(rest, (0, 2, 3, 4, 1)).reshape(-1, _Cc)
        idx_nt = _nearest(rest_nc, embT, embsq)
        idx_bthw = idx_nt.reshape(_Bb, _Tt, pn, pn)
        idx_list.append(idx_bthw)
        h = jnp.transpose(jnp.take(emb, idx_bthw, axis=0), (0, 4, 1, 2, 3))
        if si != _SN - 1:
            h = jax.image.resize(h, (_Bb, _Cc, _Tt, _Hh, _Ww),
                                 method='trilinear')
        kk = _KK[si]
        y = jax.lax.conv_general_dilated(
            h, phi_w[kk], window_strides=(1, 1, 1),
            padding=((0, 0), (1, 1), (1, 1)),
            dimension_numbers=('NCDHW', 'OIDHW', 'NCDHW'))
        y = y + phi_b[kk].reshape(1, -1, 1, 1, 1)
        h = h * (1 - ratio) + y * ratio
        f_hat = f_hat + h
        f_rest = f_rest - h
        loss = loss + jnp.mean((f_hat - f) ** 2) * _BETA + jnp.mean((f_hat - f) ** 2)
    loss = loss * (1.0 / _SN)
    return (f_hat, loss) + tuple(idx_list)
